# baseline (speedup 1.0000x reference)
import functools

import jax
import jax.numpy as jnp
from jax import lax
from jax.experimental import pallas as pl
from jax.experimental.pallas import tpu as pltpu
from jax.experimental.pallas import tpu_sc as plsc

NC = 2
NS = 16
NW = NC * NS
C = 80
NBUF = 4
DUMP = 112


@functools.lru_cache(maxsize=None)
def _make_agg(n, d, k_chunks, n_pad):
    mesh = plsc.VectorSubcoreMesh(
        core_axis_name="c", subcore_axis_name="s",
        num_cores=NC, num_subcores=NS)
    zrows = n_pad // NS
    frows = (n // NS) // 8 * 8
    tail = n - NS * frows

    scratch = (
        [pltpu.VMEM((2, C), jnp.int32) for _ in range(NBUF)]
        + [pltpu.VMEM((C, d), jnp.float32) for _ in range(NBUF)]
        + [pltpu.VMEM_SHARED((n_pad, d), jnp.float32)]
        + [pltpu.SemaphoreType.DMA for _ in range(2 * NBUF + 1)]
    )

    @functools.partial(
        pl.kernel,
        out_type=jax.ShapeDtypeStruct((NC, n, d), jnp.float32),
        mesh=mesh,
        scratch_types=scratch,
    )
    def agg(x_hbm, eidx_hbm, zeros_hbm, out_hbm, *scr):
        ibufs = scr[:NBUF]
        rbufs = scr[NBUF:2 * NBUF]
        acc_sh = scr[2 * NBUF]
        isems = scr[2 * NBUF + 1:3 * NBUF + 1]
        rsems = scr[3 * NBUF + 1:4 * NBUF + 1]
        zsem = scr[4 * NBUF + 1]
        cid = lax.axis_index("c")
        sid = lax.axis_index("s")
        wid = sid * NC + cid

        zcopy = pltpu.make_async_copy(
            zeros_hbm, acc_sh.at[pl.ds(sid * zrows, zrows)], zsem)
        zcopy.start()

        def idx_copy(j, b):
            return pltpu.make_async_copy(
                eidx_hbm.at[wid, j], ibufs[b], isems[b])

        def gather(b):
            return pltpu.make_async_copy(
                x_hbm.at[ibufs[b].at[0]], rbufs[b], rsems[b])

        for b in range(min(NBUF, k_chunks)):
            idx_copy(b, b).start()
        for b in range(min(NBUF - 1, k_chunks)):
            idx_copy(b, b).wait()
            gather(b).start()

        zcopy.wait()
        plsc.subcore_barrier()

        def chunk_group(g, carry):
            for b in range(NBUF):
                j = g * NBUF + b
                gather(b).wait()
                pltpu.sync_copy(rbufs[b], acc_sh.at[ibufs[b].at[1]], add=True)

                @pl.when(j + NBUF < k_chunks)
                def _():
                    idx_copy(j + NBUF, b).start()

                bn = (b + NBUF - 1) % NBUF

                @pl.when(j + NBUF - 1 < k_chunks)
                def _():
                    idx_copy(j + NBUF - 1, bn).wait()
                    gather(bn).start()
            return carry

        lax.fori_loop(0, k_chunks // NBUF, chunk_group, 0, unroll=False)

        plsc.subcore_barrier()
        pltpu.sync_copy(acc_sh.at[pl.ds(sid * frows, frows)],
                        out_hbm.at[cid, pl.ds(sid * frows, frows)])
        if tail:
            @pl.when(sid == 0)
            def _():
                pltpu.sync_copy(acc_sh.at[pl.ds(NS * frows, tail)],
                                out_hbm.at[cid, pl.ds(NS * frows, tail)])

    return agg


def _mlp_body(x_ref, agg_ref, wa_ref, ba_ref, wb_ref, bb_ref, scale_ref,
              o_ref, *, final_relu):
    h = scale_ref[0] * x_ref[...] + agg_ref[0] + agg_ref[1]
    t = jnp.dot(h, wa_ref[...], preferred_element_type=jnp.float32)
    t = jnp.maximum(t + ba_ref[...], 0.0)
    o = jnp.dot(t, wb_ref[...], preferred_element_type=jnp.float32)
    o = o + bb_ref[...]
    if final_relu:
        o = jnp.maximum(o, 0.0)
    o_ref[...] = o


@functools.lru_cache(maxsize=None)
def _make_mlp(n, d_in, d_hid, d_out, final_relu, bm=1000):
    grid = (n // bm,)
    return pl.pallas_call(
        functools.partial(_mlp_body, final_relu=final_relu),
        grid=grid,
        in_specs=[
            pl.BlockSpec((bm, d_in), lambda i: (i, 0)),
            pl.BlockSpec((NC, bm, d_in), lambda i: (0, i, 0)),
            pl.BlockSpec((d_in, d_hid), lambda i: (0, 0)),
            pl.BlockSpec((1, d_hid), lambda i: (0, 0)),
            pl.BlockSpec((d_hid, d_out), lambda i: (0, 0)),
            pl.BlockSpec((1, d_out), lambda i: (0, 0)),
            pl.BlockSpec(memory_space=pltpu.SMEM),
        ],
        out_specs=pl.BlockSpec((bm, d_out), lambda i: (i, 0)),
        out_shape=jax.ShapeDtypeStruct((n, d_out), jnp.float32),
    )


def kernel(x, edge_index, eps1, w11, b11, w12, b12, eps2, w21, b21, w22, b22):
    n, d = x.shape
    e = edge_index.shape[1]
    assert n % NS == 0

    k_chunks = -(-e // (NW * C))
    k_chunks += (-k_chunks) % NBUF
    e_pad = NW * k_chunks * C
    pad = e_pad - e
    n_pad = n + DUMP

    src = edge_index[0].astype(jnp.int32)
    dst = edge_index[1].astype(jnp.int32)
    if pad:
        ar = jnp.arange(pad, dtype=jnp.int32)
        src = jnp.concatenate([src, (ar * 97) % n])
        dst = jnp.concatenate([dst, n + (ar % DUMP)])
    eidx = jnp.stack(
        [src.reshape(NW, k_chunks, C), dst.reshape(NW, k_chunks, C)], axis=2)
    zrs = jnp.zeros((n_pad // NS, d), jnp.float32)

    agg_fn = _make_agg(n, d, k_chunks, n_pad)
    mlp1 = _make_mlp(n, d, w11.shape[1], w12.shape[1], True)
    mlp2 = _make_mlp(n, d, w21.shape[1], w22.shape[1], False)

    p1 = agg_fn(x, eidx, zrs)
    h = mlp1(x, p1, w11, b11.reshape(1, -1), w12, b12.reshape(1, -1),
             (1.0 + eps1).reshape(1))
    p2 = agg_fn(h, eidx, zrs)
    out = mlp2(h, p2, w21, b21.reshape(1, -1), w22, b22.reshape(1, -1),
               (1.0 + eps2).reshape(1))
    return out


# speedup vs baseline: 1.0430x; 1.0430x over previous
import functools

import jax
import jax.numpy as jnp
from jax import lax
from jax.experimental import pallas as pl
from jax.experimental.pallas import tpu as pltpu
from jax.experimental.pallas import tpu_sc as plsc

NC = 2
NS = 16
NW = NC * NS
C = 80
NBUF = 4
DUMP = 16


@functools.lru_cache(maxsize=None)
def _make_agg(n, d, k_chunks, n_pad):
    mesh = plsc.VectorSubcoreMesh(
        core_axis_name="c", subcore_axis_name="s",
        num_cores=NC, num_subcores=NS)
    zrows = n_pad // NS
    frows = (n // NS) // 8 * 8
    tail = n - NS * frows

    scratch = (
        [pltpu.VMEM((2, C), jnp.int32) for _ in range(NBUF)]
        + [pltpu.VMEM((C, d), jnp.float32) for _ in range(NBUF)]
        + [pltpu.VMEM_SHARED((n_pad, d), jnp.float32)]
        + [pltpu.SemaphoreType.DMA for _ in range(2 * NBUF)]
    )

    @functools.partial(
        pl.kernel,
        out_type=jax.ShapeDtypeStruct((NC, n, d), jnp.float32),
        mesh=mesh,
        scratch_types=scratch,
    )
    def agg(x_hbm, eidx_hbm, out_hbm, *scr):
        ibufs = scr[:NBUF]
        rbufs = scr[NBUF:2 * NBUF]
        acc_sh = scr[2 * NBUF]
        isems = scr[2 * NBUF + 1:3 * NBUF + 1]
        rsems = scr[3 * NBUF + 1:]
        cid = lax.axis_index("c")
        sid = lax.axis_index("s")
        wid = sid * NC + cid

        buf0 = rbufs[0]
        def zfill(i, carry):
            buf0[i // (d // 16), pl.ds((i % (d // 16)) * 16, 16)] = (
                jnp.zeros((16,), jnp.float32))
            return carry
        lax.fori_loop(0, C * d // 16, zfill, 0, unroll=8)
        for r0 in range(0, zrows, C):
            rr = min(C, zrows - r0)
            pltpu.sync_copy(buf0.at[pl.ds(0, rr)],
                            acc_sh.at[pl.ds(sid * zrows + r0, rr)])
        plsc.subcore_barrier()

        def idx_copy(j, b):
            return pltpu.make_async_copy(
                eidx_hbm.at[wid, j], ibufs[b], isems[b])

        def gather(b):
            return pltpu.make_async_copy(
                x_hbm.at[ibufs[b].at[0]], rbufs[b], rsems[b])

        for b in range(min(NBUF, k_chunks)):
            idx_copy(b, b).start()
        for b in range(min(NBUF - 1, k_chunks)):
            idx_copy(b, b).wait()
            gather(b).start()

        def chunk_group(g, carry):
            for b in range(NBUF):
                j = g * NBUF + b
                gather(b).wait()
                pltpu.sync_copy(rbufs[b], acc_sh.at[ibufs[b].at[1]], add=True)

                @pl.when(j + NBUF < k_chunks)
                def _():
                    idx_copy(j + NBUF, b).start()

                bn = (b + NBUF - 1) % NBUF

                @pl.when(j + NBUF - 1 < k_chunks)
                def _():
                    idx_copy(j + NBUF - 1, bn).wait()
                    gather(bn).start()
            return carry

        lax.fori_loop(0, k_chunks // NBUF, chunk_group, 0, unroll=False)

        plsc.subcore_barrier()
        pltpu.sync_copy(acc_sh.at[pl.ds(sid * frows, frows)],
                        out_hbm.at[cid, pl.ds(sid * frows, frows)])
        if tail:
            @pl.when(sid == 0)
            def _():
                pltpu.sync_copy(acc_sh.at[pl.ds(NS * frows, tail)],
                                out_hbm.at[cid, pl.ds(NS * frows, tail)])

    return agg


def _mlp_body(x_ref, agg_ref, wa_ref, ba_ref, wb_ref, bb_ref, scale_ref,
              o_ref, *, final_relu):
    h = scale_ref[0] * x_ref[...] + agg_ref[0] + agg_ref[1]
    t = jnp.dot(h, wa_ref[...], preferred_element_type=jnp.float32)
    t = jnp.maximum(t + ba_ref[...], 0.0)
    o = jnp.dot(t, wb_ref[...], preferred_element_type=jnp.float32)
    o = o + bb_ref[...]
    if final_relu:
        o = jnp.maximum(o, 0.0)
    o_ref[...] = o


@functools.lru_cache(maxsize=None)
def _make_mlp(n, d_in, d_hid, d_out, final_relu, bm=2000):
    grid = (n // bm,)
    return pl.pallas_call(
        functools.partial(_mlp_body, final_relu=final_relu),
        grid=grid,
        in_specs=[
            pl.BlockSpec((bm, d_in), lambda i: (i, 0)),
            pl.BlockSpec((NC, bm, d_in), lambda i: (0, i, 0)),
            pl.BlockSpec((d_in, d_hid), lambda i: (0, 0)),
            pl.BlockSpec((1, d_hid), lambda i: (0, 0)),
            pl.BlockSpec((d_hid, d_out), lambda i: (0, 0)),
            pl.BlockSpec((1, d_out), lambda i: (0, 0)),
            pl.BlockSpec(memory_space=pltpu.SMEM),
        ],
        out_specs=pl.BlockSpec((bm, d_out), lambda i: (i, 0)),
        out_shape=jax.ShapeDtypeStruct((n, d_out), jnp.float32),
    )


def kernel(x, edge_index, eps1, w11, b11, w12, b12, eps2, w21, b21, w22, b22):
    n, d = x.shape
    e = edge_index.shape[1]
    assert n % NS == 0

    k_chunks = -(-e // (NW * C))
    k_chunks += (-k_chunks) % NBUF
    e_pad = NW * k_chunks * C
    pad = e_pad - e
    n_pad = n + DUMP

    src = edge_index[0].astype(jnp.int32)
    dst = edge_index[1].astype(jnp.int32)
    if pad:
        ar = jnp.arange(pad, dtype=jnp.int32)
        src = jnp.concatenate([src, (ar * 97) % n])
        dst = jnp.concatenate([dst, n + (ar % DUMP)])
    eidx = jnp.stack(
        [src.reshape(NW, k_chunks, C), dst.reshape(NW, k_chunks, C)], axis=2)

    agg_fn = _make_agg(n, d, k_chunks, n_pad)
    mlp1 = _make_mlp(n, d, w11.shape[1], w12.shape[1], True)
    mlp2 = _make_mlp(n, d, w21.shape[1], w22.shape[1], False)

    p1 = agg_fn(x, eidx)
    h = mlp1(x, p1, w11, b11.reshape(1, -1), w12, b12.reshape(1, -1),
             (1.0 + eps1).reshape(1))
    p2 = agg_fn(h, eidx)
    out = mlp2(h, p2, w21, b21.reshape(1, -1), w22, b22.reshape(1, -1),
               (1.0 + eps2).reshape(1))
    return out
